# Initial kernel scaffold; baseline (speedup 1.0000x reference)
import jax
import jax.numpy as jnp
from jax.experimental import pallas as pl


def kernel(x, W1, g1, b1, W2, g2, b2):
    raise NotImplementedError("write your pallas kernel here")


# speedup vs baseline: 11.1660x; 11.1660x over previous
import functools

import jax
import jax.numpy as jnp
from jax import lax
from jax.experimental import pallas as pl
from jax.experimental.pallas import tpu as pltpu
from jax.experimental.pallas import tpu_sc as plsc

_K = 16
_ROWS = 256
_RP = 512


def _knn_body(xr_ref, xc_ref, idx_ref, *, P, R, K):
    b = pl.program_id(0)
    xr = xr_ref[0]
    xc = xc_ref[0]
    x0r, x1r, x2r = xr[:, 0:1], xr[:, 1:2], xr[:, 2:3]
    x0c, x1c, x2c = xc[0:1, :], xc[1:2, :], xc[2:3, :]
    sqr = x0r * x0r + x1r * x1r + x2r * x2r
    sqc = x0c * x0c + x1c * x1c + x2c * x2c
    dot = jnp.dot(xr, xc, preferred_element_type=jnp.float32)
    d2 = jnp.maximum(sqr + sqc - 2.0 * dot, 0.0)
    cols = lax.broadcasted_iota(jnp.int32, (R, P), 1)
    inf = jnp.float32(jnp.inf)
    picks = []
    d2m = d2
    for _ in range(K):
        m = jnp.min(d2m, axis=1, keepdims=True)
        cand = jnp.where(d2m == m, cols, jnp.int32(P))
        j = jnp.min(cand, axis=1, keepdims=True)
        picks.append(j)
        d2m = jnp.where(cols == j, inf, d2m)
    idx_ref[...] = jnp.concatenate(picks, axis=1) + b * P


def _knn_idx(x):
    B, P, _ = x.shape
    R = _ROWS
    xyz = x[..., :3]
    xr = jnp.concatenate([xyz, jnp.zeros((B, P, 5), x.dtype)], axis=-1)
    xc = jnp.transpose(xr, (0, 2, 1))
    return pl.pallas_call(
        functools.partial(_knn_body, P=P, R=R, K=_K),
        grid=(B, P // R),
        in_specs=[
            pl.BlockSpec((1, R, 8), lambda b, i: (b, i, 0)),
            pl.BlockSpec((1, 8, P), lambda b, i: (b, 0, 0)),
        ],
        out_specs=pl.BlockSpec((R, _K), lambda b, i: (b * (P // R) + i, 0)),
        out_shape=jax.ShapeDtypeStruct((B * P, _K), jnp.int32),
    )(xr, xc)


def _sc_gather(table, idx):
    N = idx.shape[0]
    D = table.shape[1]
    info = plsc.get_sparse_core_info()
    NC = info.num_cores
    NW = NC * info.num_subcores
    CH = 128
    per_w = N // NW
    n_ch = per_w // CH
    mesh = plsc.VectorSubcoreMesh(core_axis_name="c", subcore_axis_name="s")

    @functools.partial(
        pl.kernel,
        out_type=jax.ShapeDtypeStruct((N, D), jnp.float32),
        mesh=mesh,
        compiler_params=pltpu.CompilerParams(use_tc_tiling_on_sc=False),
        scratch_types=[
            pltpu.VMEM((CH,), jnp.int32),
            pltpu.VMEM((CH, D), jnp.float32),
            pltpu.SemaphoreType.DMA,
        ],
    )
    def gather(table_hbm, idx_hbm, out_hbm, idx_v, rows_v, sem):
        wid = lax.axis_index("s") * NC + lax.axis_index("c")
        base = wid * per_w

        def body(i, carry):
            off = base + i * CH
            pltpu.sync_copy(idx_hbm.at[pl.ds(off, CH)], idx_v)
            pltpu.async_copy(table_hbm.at[idx_v], rows_v, sem).wait()
            pltpu.sync_copy(rows_v, out_hbm.at[pl.ds(off, CH)])
            return carry

        lax.fori_loop(0, n_ch, body, 0)

    return gather(table, idx)


def _gelu(x):
    return 0.5 * x * (1.0 + lax.erf(x * jnp.float32(0.7071067811865476)))


def _h1_t(xb, xj_t, w1t, C):
    dx = xj_t - xb
    return (jnp.dot(xb, w1t[:C], preferred_element_type=jnp.float32)
            + jnp.dot(dx, w1t[C:], preferred_element_type=jnp.float32))


def _stats1_body(x_ref, xj_ref, w1t_ref, sum_ref, sq_ref, *, K, C):
    @pl.when(pl.program_id(0) == 0)
    def _():
        sum_ref[...] = jnp.zeros_like(sum_ref)
        sq_ref[...] = jnp.zeros_like(sq_ref)

    xb = x_ref[...]
    w1t = w1t_ref[...]
    s = jnp.zeros(sum_ref.shape, jnp.float32)
    s2 = jnp.zeros(sq_ref.shape, jnp.float32)
    for t in range(K):
        h1 = _h1_t(xb, xj_ref[t], w1t, C)
        s += jnp.sum(h1, axis=0, keepdims=True)
        s2 += jnp.sum(h1 * h1, axis=0, keepdims=True)
    sum_ref[...] += s
    sq_ref[...] += s2


def _stats2_body(x_ref, xj_ref, w1t_ref, w2t_ref, sc1_ref, sh1_ref,
                 sum_ref, sq_ref, *, K, C):
    @pl.when(pl.program_id(0) == 0)
    def _():
        sum_ref[...] = jnp.zeros_like(sum_ref)
        sq_ref[...] = jnp.zeros_like(sq_ref)

    xb = x_ref[...]
    w1t = w1t_ref[...]
    w2t = w2t_ref[...]
    sc1 = sc1_ref[...]
    sh1 = sh1_ref[...]
    s = jnp.zeros(sum_ref.shape, jnp.float32)
    s2 = jnp.zeros(sq_ref.shape, jnp.float32)
    for t in range(K):
        h1 = _h1_t(xb, xj_ref[t], w1t, C)
        a = _gelu(h1 * sc1 + sh1)
        h2 = jnp.dot(a, w2t, preferred_element_type=jnp.float32)
        s += jnp.sum(h2, axis=0, keepdims=True)
        s2 += jnp.sum(h2 * h2, axis=0, keepdims=True)
    sum_ref[...] += s
    sq_ref[...] += s2


def _final_body(x_ref, xj_ref, w1t_ref, w2t_ref, sc1_ref, sh1_ref,
                sc2_ref, sh2_ref, out_ref, *, K, C):
    xb = x_ref[...]
    w1t = w1t_ref[...]
    w2t = w2t_ref[...]
    sc1 = sc1_ref[...]
    sh1 = sh1_ref[...]
    sc2 = sc2_ref[...]
    sh2 = sh2_ref[...]
    acc = jnp.full(out_ref.shape, -jnp.inf, jnp.float32)
    for t in range(K):
        h1 = _h1_t(xb, xj_ref[t], w1t, C)
        a = _gelu(h1 * sc1 + sh1)
        h2 = jnp.dot(a, w2t, preferred_element_type=jnp.float32)
        acc = jnp.maximum(acc, _gelu(h2 * sc2 + sh2))
    out_ref[...] = acc


def _row_spec(C):
    return pl.BlockSpec((_RP, C), lambda i: (i, 0))


def _xj_spec(C):
    return pl.BlockSpec((_K, _RP, C), lambda i: (0, i, 0))


def _full_spec(shape):
    return pl.BlockSpec(shape, lambda i: tuple(0 for _ in shape))


def _stats1(x_flat, xj, w1t):
    NP, C = x_flat.shape
    O = w1t.shape[1]
    return pl.pallas_call(
        functools.partial(_stats1_body, K=_K, C=C),
        grid=(NP // _RP,),
        in_specs=[_row_spec(C), _xj_spec(C), _full_spec(w1t.shape)],
        out_specs=[_full_spec((1, O)), _full_spec((1, O))],
        out_shape=[jax.ShapeDtypeStruct((1, O), jnp.float32),
                   jax.ShapeDtypeStruct((1, O), jnp.float32)],
    )(x_flat, xj, w1t)


def _stats2(x_flat, xj, w1t, w2t, sc1, sh1):
    NP, C = x_flat.shape
    O = w2t.shape[1]
    return pl.pallas_call(
        functools.partial(_stats2_body, K=_K, C=C),
        grid=(NP // _RP,),
        in_specs=[_row_spec(C), _xj_spec(C), _full_spec(w1t.shape),
                  _full_spec(w2t.shape), _full_spec(sc1.shape),
                  _full_spec(sh1.shape)],
        out_specs=[_full_spec((1, O)), _full_spec((1, O))],
        out_shape=[jax.ShapeDtypeStruct((1, O), jnp.float32),
                   jax.ShapeDtypeStruct((1, O), jnp.float32)],
    )(x_flat, xj, w1t, w2t, sc1, sh1)


def _final(x_flat, xj, w1t, w2t, sc1, sh1, sc2, sh2):
    NP, C = x_flat.shape
    O = w2t.shape[1]
    return pl.pallas_call(
        functools.partial(_final_body, K=_K, C=C),
        grid=(NP // _RP,),
        in_specs=[_row_spec(C), _xj_spec(C), _full_spec(w1t.shape),
                  _full_spec(w2t.shape), _full_spec(sc1.shape),
                  _full_spec(sh1.shape), _full_spec(sc2.shape),
                  _full_spec(sh2.shape)],
        out_specs=pl.BlockSpec((_RP, O), lambda i: (i, 0)),
        out_shape=jax.ShapeDtypeStruct((NP, O), jnp.float32),
    )(x_flat, xj, w1t, w2t, sc1, sh1, sc2, sh2)


def kernel(x, W1, g1, b1, W2, g2, b2):
    B, P, C = x.shape
    n = jnp.float32(B * P * _K)
    eps = 1e-5

    idx = _knn_idx(x)
    idx_t = jnp.transpose(idx).reshape(-1)
    x_flat = x.reshape(B * P, C)
    xj = _sc_gather(x_flat, idx_t).reshape(_K, B * P, C)

    w1t = W1.T
    w2t = W2.T

    s1, q1 = _stats1(x_flat, xj, w1t)
    mean1 = s1 / n
    var1 = q1 / n - mean1 * mean1
    sc1 = g1[None, :] / jnp.sqrt(var1 + eps)
    sh1 = b1[None, :] - mean1 * sc1

    s2, q2 = _stats2(x_flat, xj, w1t, w2t, sc1, sh1)
    mean2 = s2 / n
    var2 = q2 / n - mean2 * mean2
    sc2 = g2[None, :] / jnp.sqrt(var2 + eps)
    sh2 = b2[None, :] - mean2 * sc2

    out = _final(x_flat, xj, w1t, w2t, sc1, sh1, sc2, sh2)
    return out.reshape(B, P, -1)


# speedup vs baseline: 13.3954x; 1.1997x over previous
import functools

import jax
import jax.numpy as jnp
from jax import lax
from jax.experimental import pallas as pl
from jax.experimental.pallas import tpu as pltpu
from jax.experimental.pallas import tpu_sc as plsc

_K = 16
_ROWS = 256
_RP = 512


def _knn_body(xr_ref, xc_ref, idx_ref, *, P, R, K):
    b = pl.program_id(0)
    xr = xr_ref[0]
    xc = xc_ref[0]
    x0r, x1r, x2r = xr[:, 0:1], xr[:, 1:2], xr[:, 2:3]
    x0c, x1c, x2c = xc[0:1, :], xc[1:2, :], xc[2:3, :]
    sqr = x0r * x0r + x1r * x1r + x2r * x2r
    sqc = x0c * x0c + x1c * x1c + x2c * x2c
    dot = jnp.dot(xr, xc, preferred_element_type=jnp.float32)
    d2 = jnp.maximum(sqr + sqc - 2.0 * dot, 0.0)
    colsf = lax.broadcasted_iota(jnp.int32, (R, P), 1).astype(jnp.float32)
    inf = jnp.float32(jnp.inf)
    big = jnp.float32(P)
    picks = []
    d2m = d2
    j = None
    for t in range(K):
        if t:
            d2m = jnp.where(colsf == j, inf, d2m)
        m = jnp.min(d2m, axis=1, keepdims=True)
        cand = jnp.where(d2m == m, colsf, big)
        j = jnp.min(cand, axis=1, keepdims=True)
        picks.append(j)
    idx = jnp.concatenate(picks, axis=1).astype(jnp.int32)
    idx_ref[...] = idx + b * P


def _knn_idx(x):
    B, P, _ = x.shape
    R = _ROWS
    xyz = x[..., :3]
    xr = jnp.concatenate([xyz, jnp.zeros((B, P, 5), x.dtype)], axis=-1)
    xc = jnp.transpose(xr, (0, 2, 1))
    return pl.pallas_call(
        functools.partial(_knn_body, P=P, R=R, K=_K),
        grid=(B, P // R),
        in_specs=[
            pl.BlockSpec((1, R, 8), lambda b, i: (b, i, 0)),
            pl.BlockSpec((1, 8, P), lambda b, i: (b, 0, 0)),
        ],
        out_specs=pl.BlockSpec((R, _K), lambda b, i: (b * (P // R) + i, 0)),
        out_shape=jax.ShapeDtypeStruct((B * P, _K), jnp.int32),
    )(xr, xc)


def _sc_gather(table, idx):
    N = idx.shape[0]
    D = table.shape[1]
    info = plsc.get_sparse_core_info()
    NC = info.num_cores
    NW = NC * info.num_subcores
    CH = 128
    per_w = N // NW
    n_ch = per_w // CH
    mesh = plsc.VectorSubcoreMesh(core_axis_name="c", subcore_axis_name="s")

    G = 8
    GW = G * CH
    n_grp = per_w // GW

    @functools.partial(
        pl.kernel,
        out_type=jax.ShapeDtypeStruct((N, D), jnp.float32),
        mesh=mesh,
        compiler_params=pltpu.CompilerParams(use_tc_tiling_on_sc=False),
        scratch_types=[
            pltpu.VMEM((GW,), jnp.int32),
            pltpu.VMEM((GW, D), jnp.float32),
            pltpu.SemaphoreType.DMA,
        ],
    )
    def gather(table_hbm, idx_hbm, out_hbm, idx_v, rows_v, sem):
        wid = lax.axis_index("s") * NC + lax.axis_index("c")
        base = wid * per_w

        def body(g, carry):
            off = base + g * GW
            pltpu.sync_copy(idx_hbm.at[pl.ds(off, GW)], idx_v)
            for b in range(G):
                pltpu.async_copy(
                    table_hbm.at[idx_v.at[pl.ds(b * CH, CH)]],
                    rows_v.at[pl.ds(b * CH, CH)], sem)
            for b in range(G):
                pltpu.make_async_copy(
                    table_hbm.at[idx_v.at[pl.ds(b * CH, CH)]],
                    rows_v.at[pl.ds(b * CH, CH)], sem).wait()
            pltpu.sync_copy(rows_v, out_hbm.at[pl.ds(off, GW)])
            return carry

        lax.fori_loop(0, n_grp, body, 0)

    return gather(table, idx)


def _gelu(x):
    return 0.5 * x * (1.0 + lax.erf(x * jnp.float32(0.7071067811865476)))


def _h1_t(xb, xj_t, w1t, C):
    dx = xj_t - xb
    return (jnp.dot(xb, w1t[:C], preferred_element_type=jnp.float32)
            + jnp.dot(dx, w1t[C:], preferred_element_type=jnp.float32))


def _stats1_body(x_ref, xj_ref, w1t_ref, sum_ref, sq_ref, *, K, C):
    @pl.when(pl.program_id(0) == 0)
    def _():
        sum_ref[...] = jnp.zeros_like(sum_ref)
        sq_ref[...] = jnp.zeros_like(sq_ref)

    xb = x_ref[...]
    w1t = w1t_ref[...]
    s = jnp.zeros(sum_ref.shape, jnp.float32)
    s2 = jnp.zeros(sq_ref.shape, jnp.float32)
    for t in range(K):
        h1 = _h1_t(xb, xj_ref[t], w1t, C)
        s += jnp.sum(h1, axis=0, keepdims=True)
        s2 += jnp.sum(h1 * h1, axis=0, keepdims=True)
    sum_ref[...] += s
    sq_ref[...] += s2


def _stats2_body(x_ref, xj_ref, w1t_ref, w2t_ref, sc1_ref, sh1_ref,
                 sum_ref, sq_ref, *, K, C):
    @pl.when(pl.program_id(0) == 0)
    def _():
        sum_ref[...] = jnp.zeros_like(sum_ref)
        sq_ref[...] = jnp.zeros_like(sq_ref)

    xb = x_ref[...]
    w1t = w1t_ref[...]
    w2t = w2t_ref[...]
    sc1 = sc1_ref[...]
    sh1 = sh1_ref[...]
    s = jnp.zeros(sum_ref.shape, jnp.float32)
    s2 = jnp.zeros(sq_ref.shape, jnp.float32)
    for t in range(K):
        h1 = _h1_t(xb, xj_ref[t], w1t, C)
        a = _gelu(h1 * sc1 + sh1)
        h2 = jnp.dot(a, w2t, preferred_element_type=jnp.float32)
        s += jnp.sum(h2, axis=0, keepdims=True)
        s2 += jnp.sum(h2 * h2, axis=0, keepdims=True)
    sum_ref[...] += s
    sq_ref[...] += s2


def _final_body(x_ref, xj_ref, w1t_ref, w2t_ref, sc1_ref, sh1_ref,
                sc2_ref, sh2_ref, out_ref, *, K, C):
    xb = x_ref[...]
    w1t = w1t_ref[...]
    w2t = w2t_ref[...]
    sc1 = sc1_ref[...]
    sh1 = sh1_ref[...]
    sc2 = sc2_ref[...]
    sh2 = sh2_ref[...]
    acc = jnp.full(out_ref.shape, -jnp.inf, jnp.float32)
    for t in range(K):
        h1 = _h1_t(xb, xj_ref[t], w1t, C)
        a = _gelu(h1 * sc1 + sh1)
        h2 = jnp.dot(a, w2t, preferred_element_type=jnp.float32)
        acc = jnp.maximum(acc, _gelu(h2 * sc2 + sh2))
    out_ref[...] = acc


def _row_spec(C):
    return pl.BlockSpec((_RP, C), lambda i: (i, 0))


def _xj_spec(C):
    return pl.BlockSpec((_K, _RP, C), lambda i: (0, i, 0))


def _full_spec(shape):
    return pl.BlockSpec(shape, lambda i: tuple(0 for _ in shape))


def _stats1(x_flat, xj, w1t):
    NP, C = x_flat.shape
    O = w1t.shape[1]
    return pl.pallas_call(
        functools.partial(_stats1_body, K=_K, C=C),
        grid=(NP // _RP,),
        in_specs=[_row_spec(C), _xj_spec(C), _full_spec(w1t.shape)],
        out_specs=[_full_spec((1, O)), _full_spec((1, O))],
        out_shape=[jax.ShapeDtypeStruct((1, O), jnp.float32),
                   jax.ShapeDtypeStruct((1, O), jnp.float32)],
    )(x_flat, xj, w1t)


def _stats2(x_flat, xj, w1t, w2t, sc1, sh1):
    NP, C = x_flat.shape
    O = w2t.shape[1]
    return pl.pallas_call(
        functools.partial(_stats2_body, K=_K, C=C),
        grid=(NP // _RP,),
        in_specs=[_row_spec(C), _xj_spec(C), _full_spec(w1t.shape),
                  _full_spec(w2t.shape), _full_spec(sc1.shape),
                  _full_spec(sh1.shape)],
        out_specs=[_full_spec((1, O)), _full_spec((1, O))],
        out_shape=[jax.ShapeDtypeStruct((1, O), jnp.float32),
                   jax.ShapeDtypeStruct((1, O), jnp.float32)],
    )(x_flat, xj, w1t, w2t, sc1, sh1)


def _final(x_flat, xj, w1t, w2t, sc1, sh1, sc2, sh2):
    NP, C = x_flat.shape
    O = w2t.shape[1]
    return pl.pallas_call(
        functools.partial(_final_body, K=_K, C=C),
        grid=(NP // _RP,),
        in_specs=[_row_spec(C), _xj_spec(C), _full_spec(w1t.shape),
                  _full_spec(w2t.shape), _full_spec(sc1.shape),
                  _full_spec(sh1.shape), _full_spec(sc2.shape),
                  _full_spec(sh2.shape)],
        out_specs=pl.BlockSpec((_RP, O), lambda i: (i, 0)),
        out_shape=jax.ShapeDtypeStruct((NP, O), jnp.float32),
    )(x_flat, xj, w1t, w2t, sc1, sh1, sc2, sh2)


def kernel(x, W1, g1, b1, W2, g2, b2):
    B, P, C = x.shape
    n = jnp.float32(B * P * _K)
    eps = 1e-5

    idx = _knn_idx(x)
    idx_t = jnp.transpose(idx).reshape(-1)
    x_flat = x.reshape(B * P, C)
    xj = _sc_gather(x_flat, idx_t).reshape(_K, B * P, C)

    w1t = W1.T
    w2t = W2.T

    s1, q1 = _stats1(x_flat, xj, w1t)
    mean1 = s1 / n
    var1 = q1 / n - mean1 * mean1
    sc1 = g1[None, :] / jnp.sqrt(var1 + eps)
    sh1 = b1[None, :] - mean1 * sc1

    s2, q2 = _stats2(x_flat, xj, w1t, w2t, sc1, sh1)
    mean2 = s2 / n
    var2 = q2 / n - mean2 * mean2
    sc2 = g2[None, :] / jnp.sqrt(var2 + eps)
    sh2 = b2[None, :] - mean2 * sc2

    out = _final(x_flat, xj, w1t, w2t, sc1, sh1, sc2, sh2)
    return out.reshape(B, P, -1)


# speedup vs baseline: 13.4317x; 1.0027x over previous
import functools

import jax
import jax.numpy as jnp
from jax import lax
from jax.experimental import pallas as pl
from jax.experimental.pallas import tpu as pltpu
from jax.experimental.pallas import tpu_sc as plsc

_K = 16
_ROWS = 256
_RP = 512


def _knn_body(xr_ref, xc_ref, idx_ref, *, P, R, K):
    b = pl.program_id(0)
    xr = xr_ref[0]
    xc = xc_ref[0]
    x0r, x1r, x2r = xr[:, 0:1], xr[:, 1:2], xr[:, 2:3]
    x0c, x1c, x2c = xc[0:1, :], xc[1:2, :], xc[2:3, :]
    sqr = x0r * x0r + x1r * x1r + x2r * x2r
    sqc = x0c * x0c + x1c * x1c + x2c * x2c
    dot = jnp.dot(xr, xc, preferred_element_type=jnp.float32)
    d2 = jnp.maximum(sqr + sqc - 2.0 * dot, 0.0)
    colsf = lax.broadcasted_iota(jnp.int32, (R, P), 1).astype(jnp.float32)
    inf = jnp.float32(jnp.inf)
    big = jnp.float32(P)
    picks = []
    d2m = d2
    j = None
    for t in range(K):
        if t:
            d2m = jnp.where(colsf == j, inf, d2m)
        m = jnp.min(d2m, axis=1, keepdims=True)
        cand = jnp.where(d2m == m, colsf, big)
        j = jnp.min(cand, axis=1, keepdims=True)
        picks.append(j)
    idx = jnp.concatenate(picks, axis=1).astype(jnp.int32)
    idx_ref[...] = idx + b * P


def _knn_idx(x):
    B, P, _ = x.shape
    R = _ROWS
    xyz = x[..., :3]
    xr = jnp.concatenate([xyz, jnp.zeros((B, P, 5), x.dtype)], axis=-1)
    xc = jnp.transpose(xr, (0, 2, 1))
    return pl.pallas_call(
        functools.partial(_knn_body, P=P, R=R, K=_K),
        grid=(B, P // R),
        in_specs=[
            pl.BlockSpec((1, R, 8), lambda b, i: (b, i, 0)),
            pl.BlockSpec((1, 8, P), lambda b, i: (b, 0, 0)),
        ],
        out_specs=pl.BlockSpec((R, _K), lambda b, i: (b * (P // R) + i, 0)),
        out_shape=jax.ShapeDtypeStruct((B * P, _K), jnp.int32),
    )(xr, xc)


def _sc_gather(table, idx):
    N = idx.shape[0]
    D = table.shape[1]
    info = plsc.get_sparse_core_info()
    NC = info.num_cores
    NW = NC * info.num_subcores
    CH = 128
    per_w = N // NW
    n_ch = per_w // CH
    mesh = plsc.VectorSubcoreMesh(core_axis_name="c", subcore_axis_name="s")

    G = 8
    GW = G * CH
    n_grp = per_w // GW

    @functools.partial(
        pl.kernel,
        out_type=jax.ShapeDtypeStruct((N, D), jnp.float32),
        mesh=mesh,
        compiler_params=pltpu.CompilerParams(use_tc_tiling_on_sc=False),
        scratch_types=[
            pltpu.VMEM((GW,), jnp.int32),
            pltpu.VMEM((GW, D), jnp.float32),
            pltpu.SemaphoreType.DMA,
        ],
    )
    def gather(table_hbm, idx_hbm, out_hbm, idx_v, rows_v, sem):
        wid = lax.axis_index("s") * NC + lax.axis_index("c")
        base = wid * per_w

        def body(g, carry):
            off = base + g * GW
            pltpu.sync_copy(idx_hbm.at[pl.ds(off, GW)], idx_v)
            for b in range(G):
                pltpu.async_copy(
                    table_hbm.at[idx_v.at[pl.ds(b * CH, CH)]],
                    rows_v.at[pl.ds(b * CH, CH)], sem)
            for b in range(G):
                pltpu.make_async_copy(
                    table_hbm.at[idx_v.at[pl.ds(b * CH, CH)]],
                    rows_v.at[pl.ds(b * CH, CH)], sem).wait()
            pltpu.sync_copy(rows_v, out_hbm.at[pl.ds(off, GW)])
            return carry

        lax.fori_loop(0, n_grp, body, 0)

    return gather(table, idx)


def _gelu(x):
    return 0.5 * x * (1.0 + lax.erf(x * jnp.float32(0.7071067811865476)))


def _h1_t(xb, xj_t, w1t, C):
    dx = xj_t - xb
    return (jnp.dot(xb, w1t[:C], preferred_element_type=jnp.float32)
            + jnp.dot(dx, w1t[C:], preferred_element_type=jnp.float32))


def _stats1_body(x_ref, xj_ref, w1t_ref, sum_ref, sq_ref, *, K, C):
    @pl.when(pl.program_id(0) == 0)
    def _():
        sum_ref[...] = jnp.zeros_like(sum_ref)
        sq_ref[...] = jnp.zeros_like(sq_ref)

    xb = x_ref[...]
    w1t = w1t_ref[...]
    s = jnp.zeros(sum_ref.shape, jnp.float32)
    s2 = jnp.zeros(sq_ref.shape, jnp.float32)
    for t in range(K):
        h1 = _h1_t(xb, xj_ref[t], w1t, C)
        s += jnp.sum(h1, axis=0, keepdims=True)
        s2 += jnp.sum(h1 * h1, axis=0, keepdims=True)
    sum_ref[...] += s
    sq_ref[...] += s2


def _bn_coeffs(s_ref, q_ref, g_ref, b_ref, n):
    mean = s_ref[...] / n
    var = q_ref[...] / n - mean * mean
    sc = g_ref[...] * lax.rsqrt(var + 1e-5)
    return sc, b_ref[...] - mean * sc


def _stats2_body(x_ref, xj_ref, w1t_ref, w2t_ref, s1_ref, q1_ref,
                 g1_ref, b1_ref, sum_ref, sq_ref, *, K, C, N):
    @pl.when(pl.program_id(0) == 0)
    def _():
        sum_ref[...] = jnp.zeros_like(sum_ref)
        sq_ref[...] = jnp.zeros_like(sq_ref)

    xb = x_ref[...]
    w1t = w1t_ref[...]
    w2t = w2t_ref[...]
    sc1, sh1 = _bn_coeffs(s1_ref, q1_ref, g1_ref, b1_ref, jnp.float32(N))
    s = jnp.zeros(sum_ref.shape, jnp.float32)
    s2 = jnp.zeros(sq_ref.shape, jnp.float32)
    for t in range(K):
        h1 = _h1_t(xb, xj_ref[t], w1t, C)
        a = _gelu(h1 * sc1 + sh1)
        h2 = jnp.dot(a, w2t, preferred_element_type=jnp.float32)
        s += jnp.sum(h2, axis=0, keepdims=True)
        s2 += jnp.sum(h2 * h2, axis=0, keepdims=True)
    sum_ref[...] += s
    sq_ref[...] += s2


def _final_body(x_ref, xj_ref, w1t_ref, w2t_ref, s1_ref, q1_ref,
                g1_ref, b1_ref, s2_ref, q2_ref, g2_ref, b2_ref,
                out_ref, *, K, C, N):
    xb = x_ref[...]
    w1t = w1t_ref[...]
    w2t = w2t_ref[...]
    sc1, sh1 = _bn_coeffs(s1_ref, q1_ref, g1_ref, b1_ref, jnp.float32(N))
    sc2, sh2 = _bn_coeffs(s2_ref, q2_ref, g2_ref, b2_ref, jnp.float32(N))
    acc = jnp.full(out_ref.shape, -jnp.inf, jnp.float32)
    for t in range(K):
        h1 = _h1_t(xb, xj_ref[t], w1t, C)
        a = _gelu(h1 * sc1 + sh1)
        h2 = jnp.dot(a, w2t, preferred_element_type=jnp.float32)
        acc = jnp.maximum(acc, _gelu(h2 * sc2 + sh2))
    out_ref[...] = acc


def _row_spec(C):
    return pl.BlockSpec((_RP, C), lambda i: (i, 0))


def _xj_spec(C):
    return pl.BlockSpec((_K, _RP, C), lambda i: (0, i, 0))


def _full_spec(shape):
    return pl.BlockSpec(shape, lambda i: tuple(0 for _ in shape))


def _stats1(x_flat, xj, w1t):
    NP, C = x_flat.shape
    O = w1t.shape[1]
    return pl.pallas_call(
        functools.partial(_stats1_body, K=_K, C=C),
        grid=(NP // _RP,),
        in_specs=[_row_spec(C), _xj_spec(C), _full_spec(w1t.shape)],
        out_specs=[_full_spec((1, O)), _full_spec((1, O))],
        out_shape=[jax.ShapeDtypeStruct((1, O), jnp.float32),
                   jax.ShapeDtypeStruct((1, O), jnp.float32)],
    )(x_flat, xj, w1t)


def _stats2(x_flat, xj, w1t, w2t, s1, q1, g1, b1):
    NP, C = x_flat.shape
    O = w2t.shape[1]
    return pl.pallas_call(
        functools.partial(_stats2_body, K=_K, C=C, N=NP * _K),
        grid=(NP // _RP,),
        in_specs=[_row_spec(C), _xj_spec(C), _full_spec(w1t.shape),
                  _full_spec(w2t.shape), _full_spec(s1.shape),
                  _full_spec(q1.shape), _full_spec(g1.shape),
                  _full_spec(b1.shape)],
        out_specs=[_full_spec((1, O)), _full_spec((1, O))],
        out_shape=[jax.ShapeDtypeStruct((1, O), jnp.float32),
                   jax.ShapeDtypeStruct((1, O), jnp.float32)],
    )(x_flat, xj, w1t, w2t, s1, q1, g1, b1)


def _final(x_flat, xj, w1t, w2t, s1, q1, g1, b1, s2, q2, g2, b2):
    NP, C = x_flat.shape
    O = w2t.shape[1]
    return pl.pallas_call(
        functools.partial(_final_body, K=_K, C=C, N=NP * _K),
        grid=(NP // _RP,),
        in_specs=[_row_spec(C), _xj_spec(C), _full_spec(w1t.shape),
                  _full_spec(w2t.shape), _full_spec(s1.shape),
                  _full_spec(q1.shape), _full_spec(g1.shape),
                  _full_spec(b1.shape), _full_spec(s2.shape),
                  _full_spec(q2.shape), _full_spec(g2.shape),
                  _full_spec(b2.shape)],
        out_specs=pl.BlockSpec((_RP, O), lambda i: (i, 0)),
        out_shape=jax.ShapeDtypeStruct((NP, O), jnp.float32),
    )(x_flat, xj, w1t, w2t, s1, q1, g1, b1, s2, q2, g2, b2)


def kernel(x, W1, g1, b1, W2, g2, b2):
    B, P, C = x.shape

    idx = _knn_idx(x)
    idx_t = jnp.transpose(idx).reshape(-1)
    x_flat = x.reshape(B * P, C)
    xj = _sc_gather(x_flat, idx_t).reshape(_K, B * P, C)

    w1t = W1.T
    w2t = W2.T
    g1r, b1r = g1[None, :], b1[None, :]
    g2r, b2r = g2[None, :], b2[None, :]

    s1, q1 = _stats1(x_flat, xj, w1t)
    s2, q2 = _stats2(x_flat, xj, w1t, w2t, s1, q1, g1r, b1r)
    out = _final(x_flat, xj, w1t, w2t, s1, q1, g1r, b1r, s2, q2, g2r, b2r)
    return out.reshape(B, P, -1)


# speedup vs baseline: 13.9817x; 1.0409x over previous
import functools

import jax
import jax.numpy as jnp
from jax import lax
from jax.experimental import pallas as pl
from jax.experimental.pallas import tpu as pltpu
from jax.experimental.pallas import tpu_sc as plsc

_K = 16
_ROWS = 256
_RP = 1024


def _knn_body(xr_ref, xc_ref, idx_ref, *, P, R, K):
    b = pl.program_id(0)
    xr = xr_ref[0]
    xc = xc_ref[0]
    x0r, x1r, x2r = xr[:, 0:1], xr[:, 1:2], xr[:, 2:3]
    x0c, x1c, x2c = xc[0:1, :], xc[1:2, :], xc[2:3, :]
    sqr = x0r * x0r + x1r * x1r + x2r * x2r
    sqc = x0c * x0c + x1c * x1c + x2c * x2c
    dot = jnp.dot(xr, xc, preferred_element_type=jnp.float32)
    d2 = jnp.maximum(sqr + sqc - 2.0 * dot, 0.0)
    colsf = lax.broadcasted_iota(jnp.int32, (R, P), 1).astype(jnp.float32)
    inf = jnp.float32(jnp.inf)
    big = jnp.float32(P)
    picks = []
    d2m = d2
    j = None
    for t in range(K):
        if t:
            d2m = jnp.where(colsf == j, inf, d2m)
        m = jnp.min(d2m, axis=1, keepdims=True)
        cand = jnp.where(d2m == m, colsf, big)
        j = jnp.min(cand, axis=1, keepdims=True)
        picks.append(j)
    idx = jnp.concatenate(picks, axis=1).astype(jnp.int32)
    idx_ref[...] = idx + b * P


def _knn_idx(x):
    B, P, _ = x.shape
    R = _ROWS
    xyz = x[..., :3]
    xr = jnp.concatenate([xyz, jnp.zeros((B, P, 5), x.dtype)], axis=-1)
    xc = jnp.transpose(xr, (0, 2, 1))
    return pl.pallas_call(
        functools.partial(_knn_body, P=P, R=R, K=_K),
        grid=(B, P // R),
        in_specs=[
            pl.BlockSpec((1, R, 8), lambda b, i: (b, i, 0)),
            pl.BlockSpec((1, 8, P), lambda b, i: (b, 0, 0)),
        ],
        out_specs=pl.BlockSpec((R, _K), lambda b, i: (b * (P // R) + i, 0)),
        out_shape=jax.ShapeDtypeStruct((B * P, _K), jnp.int32),
    )(xr, xc)


def _sc_gather(table, idx):
    N = idx.shape[0]
    D = table.shape[1]
    info = plsc.get_sparse_core_info()
    NC = info.num_cores
    NW = NC * info.num_subcores
    CH = 128
    per_w = N // NW
    n_ch = per_w // CH
    mesh = plsc.VectorSubcoreMesh(core_axis_name="c", subcore_axis_name="s")

    G = 8
    GW = G * CH
    n_grp = per_w // GW

    @functools.partial(
        pl.kernel,
        out_type=jax.ShapeDtypeStruct((N, D), jnp.float32),
        mesh=mesh,
        compiler_params=pltpu.CompilerParams(use_tc_tiling_on_sc=False),
        scratch_types=[
            pltpu.VMEM((GW,), jnp.int32),
            pltpu.VMEM((GW, D), jnp.float32),
            pltpu.SemaphoreType.DMA,
        ],
    )
    def gather(table_hbm, idx_hbm, out_hbm, idx_v, rows_v, sem):
        wid = lax.axis_index("s") * NC + lax.axis_index("c")
        base = wid * per_w

        def body(g, carry):
            off = base + g * GW
            pltpu.sync_copy(idx_hbm.at[pl.ds(off, GW)], idx_v)
            for b in range(G):
                pltpu.async_copy(
                    table_hbm.at[idx_v.at[pl.ds(b * CH, CH)]],
                    rows_v.at[pl.ds(b * CH, CH)], sem)
            for b in range(G):
                pltpu.make_async_copy(
                    table_hbm.at[idx_v.at[pl.ds(b * CH, CH)]],
                    rows_v.at[pl.ds(b * CH, CH)], sem).wait()
            pltpu.sync_copy(rows_v, out_hbm.at[pl.ds(off, GW)])
            return carry

        lax.fori_loop(0, n_grp, body, 0)

    return gather(table, idx)


def _gelu(x):
    return 0.5 * x * (1.0 + lax.erf(x * jnp.float32(0.7071067811865476)))


def _h1_t(xb, xj_t, w1t, C):
    dx = xj_t - xb
    return (jnp.dot(xb, w1t[:C], preferred_element_type=jnp.float32)
            + jnp.dot(dx, w1t[C:], preferred_element_type=jnp.float32))


def _stats1_body(x_ref, xj_ref, w1t_ref, sum_ref, sq_ref, *, K, C):
    @pl.when(pl.program_id(0) == 0)
    def _():
        sum_ref[...] = jnp.zeros_like(sum_ref)
        sq_ref[...] = jnp.zeros_like(sq_ref)

    xb = x_ref[...]
    w1t = w1t_ref[...]
    s = jnp.zeros(sum_ref.shape, jnp.float32)
    s2 = jnp.zeros(sq_ref.shape, jnp.float32)
    for t in range(K):
        h1 = _h1_t(xb, xj_ref[t], w1t, C)
        s += jnp.sum(h1, axis=0, keepdims=True)
        s2 += jnp.sum(h1 * h1, axis=0, keepdims=True)
    sum_ref[...] += s
    sq_ref[...] += s2


def _bn_coeffs(s_ref, q_ref, g_ref, b_ref, n):
    mean = s_ref[...] / n
    var = q_ref[...] / n - mean * mean
    sc = g_ref[...] * lax.rsqrt(var + 1e-5)
    return sc, b_ref[...] - mean * sc


def _stats2_body(x_ref, xj_ref, w1t_ref, w2t_ref, s1_ref, q1_ref,
                 g1_ref, b1_ref, sum_ref, sq_ref, *, K, C, N):
    @pl.when(pl.program_id(0) == 0)
    def _():
        sum_ref[...] = jnp.zeros_like(sum_ref)
        sq_ref[...] = jnp.zeros_like(sq_ref)

    xb = x_ref[...]
    w1t = w1t_ref[...]
    w2t = w2t_ref[...]
    sc1, sh1 = _bn_coeffs(s1_ref, q1_ref, g1_ref, b1_ref, jnp.float32(N))
    s = jnp.zeros(sum_ref.shape, jnp.float32)
    s2 = jnp.zeros(sq_ref.shape, jnp.float32)
    for t in range(K):
        h1 = _h1_t(xb, xj_ref[t], w1t, C)
        a = _gelu(h1 * sc1 + sh1)
        h2 = jnp.dot(a, w2t, preferred_element_type=jnp.float32)
        s += jnp.sum(h2, axis=0, keepdims=True)
        s2 += jnp.sum(h2 * h2, axis=0, keepdims=True)
    sum_ref[...] += s
    sq_ref[...] += s2


def _final_body(x_ref, xj_ref, w1t_ref, w2t_ref, s1_ref, q1_ref,
                g1_ref, b1_ref, s2_ref, q2_ref, g2_ref, b2_ref,
                out_ref, *, K, C, N):
    xb = x_ref[...]
    w1t = w1t_ref[...]
    w2t = w2t_ref[...]
    sc1, sh1 = _bn_coeffs(s1_ref, q1_ref, g1_ref, b1_ref, jnp.float32(N))
    sc2, sh2 = _bn_coeffs(s2_ref, q2_ref, g2_ref, b2_ref, jnp.float32(N))
    acc = jnp.full(out_ref.shape, -jnp.inf, jnp.float32)
    for t in range(K):
        h1 = _h1_t(xb, xj_ref[t], w1t, C)
        a = _gelu(h1 * sc1 + sh1)
        h2 = jnp.dot(a, w2t, preferred_element_type=jnp.float32)
        acc = jnp.maximum(acc, _gelu(h2 * sc2 + sh2))
    out_ref[...] = acc


def _row_spec(C):
    return pl.BlockSpec((_RP, C), lambda i: (i, 0))


def _xj_spec(C):
    return pl.BlockSpec((_K, _RP, C), lambda i: (0, i, 0))


def _full_spec(shape):
    return pl.BlockSpec(shape, lambda i: tuple(0 for _ in shape))


def _stats1(x_flat, xj, w1t):
    NP, C = x_flat.shape
    O = w1t.shape[1]
    return pl.pallas_call(
        functools.partial(_stats1_body, K=_K, C=C),
        grid=(NP // _RP,),
        in_specs=[_row_spec(C), _xj_spec(C), _full_spec(w1t.shape)],
        out_specs=[_full_spec((1, O)), _full_spec((1, O))],
        out_shape=[jax.ShapeDtypeStruct((1, O), jnp.float32),
                   jax.ShapeDtypeStruct((1, O), jnp.float32)],
    )(x_flat, xj, w1t)


def _stats2(x_flat, xj, w1t, w2t, s1, q1, g1, b1):
    NP, C = x_flat.shape
    O = w2t.shape[1]
    return pl.pallas_call(
        functools.partial(_stats2_body, K=_K, C=C, N=NP * _K),
        grid=(NP // _RP,),
        in_specs=[_row_spec(C), _xj_spec(C), _full_spec(w1t.shape),
                  _full_spec(w2t.shape), _full_spec(s1.shape),
                  _full_spec(q1.shape), _full_spec(g1.shape),
                  _full_spec(b1.shape)],
        out_specs=[_full_spec((1, O)), _full_spec((1, O))],
        out_shape=[jax.ShapeDtypeStruct((1, O), jnp.float32),
                   jax.ShapeDtypeStruct((1, O), jnp.float32)],
    )(x_flat, xj, w1t, w2t, s1, q1, g1, b1)


def _final(x_flat, xj, w1t, w2t, s1, q1, g1, b1, s2, q2, g2, b2):
    NP, C = x_flat.shape
    O = w2t.shape[1]
    return pl.pallas_call(
        functools.partial(_final_body, K=_K, C=C, N=NP * _K),
        grid=(NP // _RP,),
        in_specs=[_row_spec(C), _xj_spec(C), _full_spec(w1t.shape),
                  _full_spec(w2t.shape), _full_spec(s1.shape),
                  _full_spec(q1.shape), _full_spec(g1.shape),
                  _full_spec(b1.shape), _full_spec(s2.shape),
                  _full_spec(q2.shape), _full_spec(g2.shape),
                  _full_spec(b2.shape)],
        out_specs=pl.BlockSpec((_RP, O), lambda i: (i, 0)),
        out_shape=jax.ShapeDtypeStruct((NP, O), jnp.float32),
    )(x_flat, xj, w1t, w2t, s1, q1, g1, b1, s2, q2, g2, b2)


def kernel(x, W1, g1, b1, W2, g2, b2):
    B, P, C = x.shape

    idx = _knn_idx(x)
    idx_t = jnp.transpose(idx).reshape(-1)
    x_flat = x.reshape(B * P, C)
    xj = _sc_gather(x_flat, idx_t).reshape(_K, B * P, C)

    w1t = W1.T
    w2t = W2.T
    g1r, b1r = g1[None, :], b1[None, :]
    g2r, b2r = g2[None, :], b2[None, :]

    s1, q1 = _stats1(x_flat, xj, w1t)
    s2, q2 = _stats2(x_flat, xj, w1t, w2t, s1, q1, g1r, b1r)
    out = _final(x_flat, xj, w1t, w2t, s1, q1, g1r, b1r, s2, q2, g2r, b2r)
    return out.reshape(B, P, -1)
